# Initial kernel scaffold; baseline (speedup 1.0000x reference)
import jax
import jax.numpy as jnp
from jax.experimental import pallas as pl


def kernel(x, pred_class, W0, b0, g0, be0, W1, b1, g1, be1, W2, b2, g2, be2, W3, b3):
    raise NotImplementedError("write your pallas kernel here")


# speedup vs baseline: 9269.4095x; 9269.4095x over previous
import functools

import numpy as np

import jax
import jax.numpy as jnp
from jax.experimental import pallas as pl

IN = 128
NB = 512
EPS = 1e-5
NEWTON_ITERS = 6
HALLEY_ITERS = 1
C232 = 232


def _mm_t(a, b):
    return jax.lax.dot_general(
        a, b, (((0,), (0,)), ((), ())), preferred_element_type=jnp.float32)


def _bn_scale_shift(st, g, be, n):
    mu = st[:, 0:1] * (1.0 / n)
    var = st[:, 1:2] * (1.0 / n) - mu * mu
    rstd = jax.lax.rsqrt(var + EPS)
    scale = g * rstd
    shift = be - mu * scale
    return scale, shift


def _accum_stats(y, st_ref):
    s1 = jnp.sum(y, axis=1, keepdims=True)
    s2 = jnp.sum(y * y, axis=1, keepdims=True)
    part = jnp.concatenate([s1, s2], axis=1)
    pid = pl.program_id(0)

    @pl.when(pid == 0)
    def _():
        st_ref[...] = part

    @pl.when(pid != 0)
    def _():
        st_ref[...] += part


def _k_first(xt_ref, w_ref, b_ref, y_ref, st_ref):
    y = _mm_t(w_ref[...], xt_ref[...]) + b_ref[...]
    y_ref[...] = y
    _accum_stats(y, st_ref)


def _k_mid(n, y_in_ref, st_in_ref, g_ref, be_ref, w_ref, b_ref, y_ref, st_ref):
    scale, shift = _bn_scale_shift(st_in_ref[...], g_ref[...], be_ref[...], n)
    h = jnp.maximum(y_in_ref[...] * scale + shift, 0.0)
    y = _mm_t(w_ref[...], h) + b_ref[...]
    y_ref[...] = y
    _accum_stats(y, st_ref)


def _newton_step(x):
    c00 = x[4] * x[8] - x[5] * x[7]
    c01 = x[5] * x[6] - x[3] * x[8]
    c02 = x[3] * x[7] - x[4] * x[6]
    c10 = x[2] * x[7] - x[1] * x[8]
    c11 = x[0] * x[8] - x[2] * x[6]
    c12 = x[1] * x[6] - x[0] * x[7]
    c20 = x[1] * x[5] - x[2] * x[4]
    c21 = x[2] * x[3] - x[0] * x[5]
    c22 = x[0] * x[4] - x[1] * x[3]
    det = x[0] * c00 + x[1] * c01 + x[2] * c02
    sgn = jnp.where(det < 0, -1.0, 1.0)
    absdet = jnp.maximum(jnp.abs(det), 1e-9)
    zeta = jnp.minimum(jnp.exp(jnp.log(absdet) * (-1.0 / 3.0)), 1e3)
    w = sgn / (zeta * absdet)
    c = [c00, c01, c02, c10, c11, c12, c20, c21, c22]
    return [0.5 * (zeta * xi + w * ci) for xi, ci in zip(x, c)]


def _halley_step(x):
    a00 = x[0] * x[0] + x[3] * x[3] + x[6] * x[6]
    a01 = x[0] * x[1] + x[3] * x[4] + x[6] * x[7]
    a02 = x[0] * x[2] + x[3] * x[5] + x[6] * x[8]
    a11 = x[1] * x[1] + x[4] * x[4] + x[7] * x[7]
    a12 = x[1] * x[2] + x[4] * x[5] + x[7] * x[8]
    a22 = x[2] * x[2] + x[5] * x[5] + x[8] * x[8]
    d00, d01, d02 = 1.0 + 3.0 * a00, 3.0 * a01, 3.0 * a02
    d11, d12, d22 = 1.0 + 3.0 * a11, 3.0 * a12, 1.0 + 3.0 * a22
    j00 = d11 * d22 - d12 * d12
    j01 = d02 * d12 - d01 * d22
    j02 = d01 * d12 - d02 * d11
    j11 = d00 * d22 - d02 * d02
    j12 = d01 * d02 - d00 * d12
    j22 = d00 * d11 - d01 * d01
    detd = d00 * j00 + d01 * j01 + d02 * j02
    rdetd = 1.0 / detd
    b00, b01, b02 = 3.0 + a00, a01, a02
    b11, b12, b22 = 3.0 + a11, a12, 3.0 + a22
    m00 = b00 * j00 + b01 * j01 + b02 * j02
    m01 = b00 * j01 + b01 * j11 + b02 * j12
    m02 = b00 * j02 + b01 * j12 + b02 * j22
    m10 = b01 * j00 + b11 * j01 + b12 * j02
    m11 = b01 * j01 + b11 * j11 + b12 * j12
    m12 = b01 * j02 + b11 * j12 + b12 * j22
    m20 = b02 * j00 + b12 * j01 + b22 * j02
    m21 = b02 * j01 + b12 * j11 + b22 * j12
    m22 = b02 * j02 + b12 * j12 + b22 * j22
    return [
        (x[0] * m00 + x[1] * m10 + x[2] * m20) * rdetd,
        (x[0] * m01 + x[1] * m11 + x[2] * m21) * rdetd,
        (x[0] * m02 + x[1] * m12 + x[2] * m22) * rdetd,
        (x[3] * m00 + x[4] * m10 + x[5] * m20) * rdetd,
        (x[3] * m01 + x[4] * m11 + x[5] * m21) * rdetd,
        (x[3] * m02 + x[4] * m12 + x[5] * m22) * rdetd,
        (x[6] * m00 + x[7] * m10 + x[8] * m20) * rdetd,
        (x[6] * m01 + x[7] * m11 + x[8] * m21) * rdetd,
        (x[6] * m02 + x[7] * m12 + x[8] * m22) * rdetd,
    ]


def _k_head(n, y2_ref, st2_ref, g2_ref, be2_ref, y0_ref, st0_ref, g0_ref,
            be0_ref, w3_ref, b3_ref, p2_ref, rot_ref, aux_ref):
    sc0, sh0 = _bn_scale_shift(st0_ref[...], g0_ref[...], be0_ref[...], n)
    h = jnp.maximum(y0_ref[...] * sc0 + sh0, 0.0)
    sc2, sh2 = _bn_scale_shift(st2_ref[...], g2_ref[...], be2_ref[...], n)
    r = y2_ref[...] * sc2 + sh2
    h2 = jnp.maximum(h + r, 0.0)
    ot = _mm_t(w3_ref[...], h2) + b3_ref[...]

    m = [ot[24 * e:24 * e + 24, :] for e in range(9)]
    d0 = (m[0] * (m[4] * m[8] - m[5] * m[7])
          - m[1] * (m[3] * m[8] - m[5] * m[6])
          + m[2] * (m[3] * m[7] - m[4] * m[6]))
    sgn0 = jnp.where(d0 < 0, -1.0, 1.0)
    fn2 = sum(mi * mi for mi in m)
    rfn = jax.lax.rsqrt(fn2 + 1e-30)
    x = [mi * rfn for mi in m]
    for _ in range(NEWTON_ITERS):
        x = _newton_step(x)
    for _ in range(HALLEY_ITERS):
        x = _halley_step(x)
    x = [xi * sgn0 for xi in x]

    rt = jnp.concatenate(x + [ot[216:232, :]], axis=0)
    outn = _mm_t(rt, p2_ref[...])
    rot_ref[...] = outn[:, :216]
    aux_ref[...] = outn[:, 216:232]


def _head_constants(W3, b3):
    src = np.array([k * 9 + e for e in range(9) for k in range(24)]
                   + list(range(216, 229)), dtype=np.int32)
    w3p = jnp.concatenate(
        [W3[:, src], jnp.zeros((IN, C232 - 229), jnp.float32)], axis=1)
    b3p = jnp.concatenate(
        [b3[src], jnp.zeros((C232 - 229,), jnp.float32)]).reshape(C232, 1)
    p2 = np.zeros((C232, C232), np.float32)
    for e in range(9):
        for k in range(24):
            p2[e * 24 + k, k * 9 + e] = 1.0
    for j in range(13):
        p2[216 + j, 216 + j] = 1.0
    return w3p, b3p, jnp.asarray(p2)


def kernel(x, pred_class, W0, b0, g0, be0, W1, b1, g1, be1, W2, b2, g2, be2,
           W3, b3):
    stage, bs, nq, ch = x.shape
    n = stage * bs * nq
    grid = n // NB
    del pred_class

    xt = x.reshape(n, ch).T
    w3p, b3p, p2 = _head_constants(W3, b3)
    col = lambda v: v.reshape(ch, 1)

    act_spec = pl.BlockSpec((IN, NB), lambda i: (0, i))
    st_spec = pl.BlockSpec((IN, 2), lambda i: (0, 0))
    w_spec = pl.BlockSpec((IN, IN), lambda i: (0, 0))
    vec_spec = pl.BlockSpec((IN, 1), lambda i: (0, 0))
    act_shape = jax.ShapeDtypeStruct((IN, n), jnp.float32)
    st_shape = jax.ShapeDtypeStruct((IN, 2), jnp.float32)

    y0, st0 = pl.pallas_call(
        _k_first,
        grid=(grid,),
        in_specs=[act_spec, w_spec, vec_spec],
        out_specs=[act_spec, st_spec],
        out_shape=[act_shape, st_shape],
    )(xt, W0, col(b0))

    mid = functools.partial(_k_mid, n)
    y1, st1 = pl.pallas_call(
        mid,
        grid=(grid,),
        in_specs=[act_spec, st_spec, vec_spec, vec_spec, w_spec, vec_spec],
        out_specs=[act_spec, st_spec],
        out_shape=[act_shape, st_shape],
    )(y0, st0, col(g0), col(be0), W1, col(b1))

    y2, st2 = pl.pallas_call(
        mid,
        grid=(grid,),
        in_specs=[act_spec, st_spec, vec_spec, vec_spec, w_spec, vec_spec],
        out_specs=[act_spec, st_spec],
        out_shape=[act_shape, st_shape],
    )(y1, st1, col(g1), col(be1), W2, col(b2))

    rot, aux = pl.pallas_call(
        functools.partial(_k_head, n),
        grid=(grid,),
        in_specs=[
            act_spec, st_spec, vec_spec, vec_spec,
            act_spec, st_spec, vec_spec, vec_spec,
            pl.BlockSpec((IN, C232), lambda i: (0, 0)),
            pl.BlockSpec((C232, 1), lambda i: (0, 0)),
            pl.BlockSpec((C232, C232), lambda i: (0, 0)),
        ],
        out_specs=[
            pl.BlockSpec((NB, 216), lambda i: (i, 0)),
            pl.BlockSpec((NB, 16), lambda i: (i, 0)),
        ],
        out_shape=[
            jax.ShapeDtypeStruct((n, 216), jnp.float32),
            jax.ShapeDtypeStruct((n, 16), jnp.float32),
        ],
    )(y2, st2, col(g2), col(be2), y0, st0, col(g0), col(be0), w3p, b3p, p2)

    rotmat = rot.reshape(stage, bs, nq, 24, 3, 3)
    betas = aux[:, :10].reshape(stage, bs, nq, 10)
    camera = aux[:, 10:13].reshape(stage, bs, nq, 3)
    return rotmat, betas, camera


# speedup vs baseline: 10184.8715x; 1.0988x over previous
import functools

import numpy as np

import jax
import jax.numpy as jnp
from jax.experimental import pallas as pl

IN = 128
NB = 512
EPS = 1e-5
NEWTON_ITERS = 4
HALLEY_ITERS = 1
C232 = 232


def _mm_t(a, b):
    return jax.lax.dot_general(
        a, b, (((0,), (0,)), ((), ())), preferred_element_type=jnp.float32)


def _bn_scale_shift(st, g, be, n):
    mu = st[:, 0:1] * (1.0 / n)
    var = st[:, 1:2] * (1.0 / n) - mu * mu
    rstd = jax.lax.rsqrt(var + EPS)
    scale = g * rstd
    shift = be - mu * scale
    return scale, shift


def _accum_stats(y, st_ref):
    s1 = jnp.sum(y, axis=1, keepdims=True)
    s2 = jnp.sum(y * y, axis=1, keepdims=True)
    part = jnp.concatenate([s1, s2], axis=1)
    pid = pl.program_id(0)

    @pl.when(pid == 0)
    def _():
        st_ref[...] = part

    @pl.when(pid != 0)
    def _():
        st_ref[...] += part


def _k_first(x_ref, w_ref, b_ref, y_ref, st_ref):
    y = jax.lax.dot_general(
        w_ref[...], x_ref[...], (((0,), (1,)), ((), ())),
        preferred_element_type=jnp.float32) + b_ref[...]
    y_ref[...] = y
    _accum_stats(y, st_ref)


def _k_mid(n, y_in_ref, st_in_ref, g_ref, be_ref, w_ref, b_ref, y_ref, st_ref):
    scale, shift = _bn_scale_shift(st_in_ref[...], g_ref[...], be_ref[...], n)
    h = jnp.maximum(y_in_ref[...] * scale + shift, 0.0)
    y = _mm_t(w_ref[...], h) + b_ref[...]
    y_ref[...] = y
    _accum_stats(y, st_ref)


def _newton_step(x):
    c00 = x[4] * x[8] - x[5] * x[7]
    c01 = x[5] * x[6] - x[3] * x[8]
    c02 = x[3] * x[7] - x[4] * x[6]
    c10 = x[2] * x[7] - x[1] * x[8]
    c11 = x[0] * x[8] - x[2] * x[6]
    c12 = x[1] * x[6] - x[0] * x[7]
    c20 = x[1] * x[5] - x[2] * x[4]
    c21 = x[2] * x[3] - x[0] * x[5]
    c22 = x[0] * x[4] - x[1] * x[3]
    det = x[0] * c00 + x[1] * c01 + x[2] * c02
    sgn = jnp.where(det < 0, -1.0, 1.0)
    absdet = jnp.maximum(jnp.abs(det), 1e-9)
    zeta = jnp.minimum(jnp.exp(jnp.log(absdet) * (-1.0 / 3.0)), 1e3)
    w = sgn / (zeta * absdet)
    c = [c00, c01, c02, c10, c11, c12, c20, c21, c22]
    return [0.5 * (zeta * xi + w * ci) for xi, ci in zip(x, c)]


def _halley_step(x):
    a00 = x[0] * x[0] + x[3] * x[3] + x[6] * x[6]
    a01 = x[0] * x[1] + x[3] * x[4] + x[6] * x[7]
    a02 = x[0] * x[2] + x[3] * x[5] + x[6] * x[8]
    a11 = x[1] * x[1] + x[4] * x[4] + x[7] * x[7]
    a12 = x[1] * x[2] + x[4] * x[5] + x[7] * x[8]
    a22 = x[2] * x[2] + x[5] * x[5] + x[8] * x[8]
    d00, d01, d02 = 1.0 + 3.0 * a00, 3.0 * a01, 3.0 * a02
    d11, d12, d22 = 1.0 + 3.0 * a11, 3.0 * a12, 1.0 + 3.0 * a22
    j00 = d11 * d22 - d12 * d12
    j01 = d02 * d12 - d01 * d22
    j02 = d01 * d12 - d02 * d11
    j11 = d00 * d22 - d02 * d02
    j12 = d01 * d02 - d00 * d12
    j22 = d00 * d11 - d01 * d01
    detd = d00 * j00 + d01 * j01 + d02 * j02
    rdetd = 1.0 / detd
    b00, b01, b02 = 3.0 + a00, a01, a02
    b11, b12, b22 = 3.0 + a11, a12, 3.0 + a22
    m00 = b00 * j00 + b01 * j01 + b02 * j02
    m01 = b00 * j01 + b01 * j11 + b02 * j12
    m02 = b00 * j02 + b01 * j12 + b02 * j22
    m10 = b01 * j00 + b11 * j01 + b12 * j02
    m11 = b01 * j01 + b11 * j11 + b12 * j12
    m12 = b01 * j02 + b11 * j12 + b12 * j22
    m20 = b02 * j00 + b12 * j01 + b22 * j02
    m21 = b02 * j01 + b12 * j11 + b22 * j12
    m22 = b02 * j02 + b12 * j12 + b22 * j22
    return [
        (x[0] * m00 + x[1] * m10 + x[2] * m20) * rdetd,
        (x[0] * m01 + x[1] * m11 + x[2] * m21) * rdetd,
        (x[0] * m02 + x[1] * m12 + x[2] * m22) * rdetd,
        (x[3] * m00 + x[4] * m10 + x[5] * m20) * rdetd,
        (x[3] * m01 + x[4] * m11 + x[5] * m21) * rdetd,
        (x[3] * m02 + x[4] * m12 + x[5] * m22) * rdetd,
        (x[6] * m00 + x[7] * m10 + x[8] * m20) * rdetd,
        (x[6] * m01 + x[7] * m11 + x[8] * m21) * rdetd,
        (x[6] * m02 + x[7] * m12 + x[8] * m22) * rdetd,
    ]


def _k_head(n, y2_ref, st2_ref, g2_ref, be2_ref, y0_ref, st0_ref, g0_ref,
            be0_ref, w3_ref, b3_ref, p2_ref, rot_ref, bet_ref, cam_ref):
    sc0, sh0 = _bn_scale_shift(st0_ref[...], g0_ref[...], be0_ref[...], n)
    h = jnp.maximum(y0_ref[...] * sc0 + sh0, 0.0)
    sc2, sh2 = _bn_scale_shift(st2_ref[...], g2_ref[...], be2_ref[...], n)
    r = y2_ref[...] * sc2 + sh2
    h2 = jnp.maximum(h + r, 0.0)
    ot = _mm_t(w3_ref[...], h2) + b3_ref[...]

    m = [ot[24 * e:24 * e + 24, :] for e in range(9)]
    d0 = (m[0] * (m[4] * m[8] - m[5] * m[7])
          - m[1] * (m[3] * m[8] - m[5] * m[6])
          + m[2] * (m[3] * m[7] - m[4] * m[6]))
    sgn0 = jnp.where(d0 < 0, -1.0, 1.0)
    fn2 = sum(mi * mi for mi in m)
    rfn = jax.lax.rsqrt(fn2 + 1e-30)
    x = [mi * rfn for mi in m]
    for _ in range(NEWTON_ITERS):
        x = _newton_step(x)
    for _ in range(HALLEY_ITERS):
        x = _halley_step(x)
    x = [xi * sgn0 for xi in x]

    rt = jnp.concatenate(x + [ot[216:232, :]], axis=0)
    outn = _mm_t(rt, p2_ref[...])
    rot_ref[...] = outn[:, :216]
    bet_ref[...] = outn[:, 216:226]
    cam_ref[...] = outn[:, 226:229]


def _head_constants(W3, b3):
    src = np.array([k * 9 + e for e in range(9) for k in range(24)]
                   + list(range(216, 229)), dtype=np.int32)
    w3p = jnp.concatenate(
        [W3[:, src], jnp.zeros((IN, C232 - 229), jnp.float32)], axis=1)
    b3p = jnp.concatenate(
        [b3[src], jnp.zeros((C232 - 229,), jnp.float32)]).reshape(C232, 1)
    p2 = np.zeros((C232, C232), np.float32)
    for e in range(9):
        for k in range(24):
            p2[e * 24 + k, k * 9 + e] = 1.0
    for j in range(13):
        p2[216 + j, 216 + j] = 1.0
    return w3p, b3p, jnp.asarray(p2)


def kernel(x, pred_class, W0, b0, g0, be0, W1, b1, g1, be1, W2, b2, g2, be2,
           W3, b3):
    stage, bs, nq, ch = x.shape
    n = stage * bs * nq
    grid = n // NB
    del pred_class

    xf = x.reshape(n, ch)
    w3p, b3p, p2 = _head_constants(W3, b3)
    col = lambda v: v.reshape(ch, 1)

    act_spec = pl.BlockSpec((IN, NB), lambda i: (0, i))
    st_spec = pl.BlockSpec((IN, 2), lambda i: (0, 0))
    w_spec = pl.BlockSpec((IN, IN), lambda i: (0, 0))
    vec_spec = pl.BlockSpec((IN, 1), lambda i: (0, 0))
    act_shape = jax.ShapeDtypeStruct((IN, n), jnp.float32)
    st_shape = jax.ShapeDtypeStruct((IN, 2), jnp.float32)

    y0, st0 = pl.pallas_call(
        _k_first,
        grid=(grid,),
        in_specs=[pl.BlockSpec((NB, IN), lambda i: (i, 0)), w_spec, vec_spec],
        out_specs=[act_spec, st_spec],
        out_shape=[act_shape, st_shape],
    )(xf, W0, col(b0))

    mid = functools.partial(_k_mid, n)
    y1, st1 = pl.pallas_call(
        mid,
        grid=(grid,),
        in_specs=[act_spec, st_spec, vec_spec, vec_spec, w_spec, vec_spec],
        out_specs=[act_spec, st_spec],
        out_shape=[act_shape, st_shape],
    )(y0, st0, col(g0), col(be0), W1, col(b1))

    y2, st2 = pl.pallas_call(
        mid,
        grid=(grid,),
        in_specs=[act_spec, st_spec, vec_spec, vec_spec, w_spec, vec_spec],
        out_specs=[act_spec, st_spec],
        out_shape=[act_shape, st_shape],
    )(y1, st1, col(g1), col(be1), W2, col(b2))

    rot, bet, cam = pl.pallas_call(
        functools.partial(_k_head, n),
        grid=(grid,),
        in_specs=[
            act_spec, st_spec, vec_spec, vec_spec,
            act_spec, st_spec, vec_spec, vec_spec,
            pl.BlockSpec((IN, C232), lambda i: (0, 0)),
            pl.BlockSpec((C232, 1), lambda i: (0, 0)),
            pl.BlockSpec((C232, C232), lambda i: (0, 0)),
        ],
        out_specs=[
            pl.BlockSpec((NB, 216), lambda i: (i, 0)),
            pl.BlockSpec((NB, 10), lambda i: (i, 0)),
            pl.BlockSpec((NB, 3), lambda i: (i, 0)),
        ],
        out_shape=[
            jax.ShapeDtypeStruct((n, 216), jnp.float32),
            jax.ShapeDtypeStruct((n, 10), jnp.float32),
            jax.ShapeDtypeStruct((n, 3), jnp.float32),
        ],
    )(y2, st2, col(g2), col(be2), y0, st0, col(g0), col(be0), w3p, b3p, p2)

    rotmat = rot.reshape(stage, bs, nq, 24, 3, 3)
    betas = bet.reshape(stage, bs, nq, 10)
    camera = cam.reshape(stage, bs, nq, 3)
    return rotmat, betas, camera
